# baseline (speedup 1.0000x reference)
import functools

import jax
import jax.numpy as jnp
from jax import lax
from jax.experimental import pallas as pl
from jax.experimental.pallas import tpu as pltpu
from jax.experimental.pallas import tpu_sc as plsc

_PAD_ID = 0
_REG1 = 0.01
_REG2 = 0.1
_CH = 128
_NCHUNK = 2
_BB = 128


def _gather_rows_sc(table, idx3):
    nw, nch, ch = idx3.shape
    _, d = table.shape
    n = nw * nch * ch
    info = plsc.get_sparse_core_info()
    nc = info.num_cores
    mesh = plsc.VectorSubcoreMesh(core_axis_name="c", subcore_axis_name="s")

    @functools.partial(
        pl.kernel,
        out_type=jax.ShapeDtypeStruct((n, d), jnp.float32),
        mesh=mesh,
        scratch_types=[
            pltpu.VMEM((nch, ch), jnp.int32),
            pltpu.VMEM((ch, d), jnp.float32),
            pltpu.VMEM((ch, d), jnp.float32),
            pltpu.SemaphoreType.DMA,
            pltpu.SemaphoreType.DMA,
        ],
    )
    def k(tab_hbm, idx_hbm, out_hbm, idx_v, buf0, buf1, sem0, sem1):
        wid = lax.axis_index("s") * nc + lax.axis_index("c")
        base = wid * (nch * ch)
        pltpu.sync_copy(idx_hbm.at[wid], idx_v)
        pltpu.async_copy(tab_hbm.at[idx_v.at[0]], buf0, sem0)

        def body(jj, carry):
            j0 = 2 * jj
            pltpu.async_copy(tab_hbm.at[idx_v.at[j0 + 1]], buf1, sem1)
            pltpu.make_async_copy(tab_hbm.at[idx_v.at[j0]], buf0, sem0).wait()
            pltpu.sync_copy(buf0, out_hbm.at[pl.ds(base + j0 * ch, ch)])

            @pl.when(2 * jj + 2 < nch)
            def _():
                pltpu.async_copy(tab_hbm.at[idx_v.at[j0 + 2]], buf0, sem0)

            pltpu.make_async_copy(tab_hbm.at[idx_v.at[j0 + 1]], buf1, sem1).wait()
            pltpu.sync_copy(buf1, out_hbm.at[pl.ds(base + (j0 + 1) * ch, ch)])
            return carry

        lax.fori_loop(0, nch // 2, body, 0)
        if nch % 2 == 1:
            jl = nch - 1
            pltpu.make_async_copy(tab_hbm.at[idx_v.at[jl]], buf0, sem0).wait()
            pltpu.sync_copy(buf0, out_hbm.at[pl.ds(base + jl * ch, ch)])

    return k(table, idx3)


def _loss_tc_chunk(p2d, g3, target, acc_in, z_in, c, bc, d, s, last):
    bb = _BB
    grid = bc // bb
    goff = c * grid

    def body(p_ref, g_ref, t_ref, ai_ref, zi_ref, out_ref, z_ref):
        i = pl.program_id(0)
        pb = p_ref[...].astype(jnp.bfloat16).reshape(bb, d, s)
        gg = g_ref[...]
        t = t_ref[...]
        m = (t != _PAD_ID).astype(jnp.float32)
        pf = pb.astype(jnp.float32)
        z2 = jnp.sum(pf * pf, axis=1)
        z = jnp.sqrt(z2)
        v = jnp.float32(float(d))
        sq = jnp.sqrt((v + 1.0) ** 2 + z2)
        lc = sq - (v - 1.0) * jnp.log((v - 1.0) + sq)
        base = -lc + _REG1 * z

        part = jnp.sum(base * m)
        cnt = jnp.sum(m)

        gm = (gg * m[..., None]).astype(jnp.bfloat16)

        zp0 = lax.dot_general(
            pb[0], gm[0], (((1,), (0,)), ((), ())),
            preferred_element_type=jnp.float32,
        )
        zp1 = lax.dot_general(
            pb[1], gm[1], (((1,), (0,)), ((), ())),
            preferred_element_type=jnp.float32,
        )
        for j in range(2, bb, 2):
            zp0 = zp0 + lax.dot_general(
                pb[j], gm[j], (((1,), (0,)), ((), ())),
                preferred_element_type=jnp.float32,
            )
            zp1 = zp1 + lax.dot_general(
                pb[j + 1], gm[j + 1], (((1,), (0,)), ((), ())),
                preferred_element_type=jnp.float32,
            )
        zstep = zp0 + zp1

        lane = lax.broadcasted_iota(jnp.int32, (1, 128), 1)
        upd = jnp.where(lane == 0, part, 0.0) + jnp.where(lane == 1, cnt, 0.0)

        @pl.when(i == 0)
        def _init():
            out_ref[...] = ai_ref[...]
            z_ref[...] = zi_ref[...]

        acc = out_ref[...] + upd
        out_ref[...] = acc
        zacc = z_ref[...] + zstep
        z_ref[...] = zacc

        if last:
            @pl.when(i == grid - 1)
            def _fin():
                eye = (
                    lax.broadcasted_iota(jnp.int32, (d, d), 0)
                    == lax.broadcasted_iota(jnp.int32, (d, d), 1)
                ).astype(jnp.float32)
                dotsum = jnp.sum(zacc * eye)
                tot = jnp.sum(jnp.where(lane == 0, acc, 0.0)) - _REG2 * dotsum
                den = jnp.sum(jnp.where(lane == 1, acc, 0.0))
                out_ref[...] = jnp.full((1, 128), tot / den, jnp.float32)

    return pl.pallas_call(
        body,
        grid=(grid,),
        in_specs=[
            pl.BlockSpec((bb, d * s), lambda i: (goff + i, 0)),
            pl.BlockSpec((bb, s, d), lambda i: (i, 0, 0)),
            pl.BlockSpec((bb, s), lambda i: (goff + i, 0)),
            pl.BlockSpec((1, 128), lambda i: (0, 0)),
            pl.BlockSpec((d, d), lambda i: (0, 0)),
        ],
        out_specs=[
            pl.BlockSpec((1, 128), lambda i: (0, 0)),
            pl.BlockSpec((d, d), lambda i: (0, 0)),
        ],
        out_shape=[
            jax.ShapeDtypeStruct((1, 128), jnp.float32),
            jax.ShapeDtypeStruct((d, d), jnp.float32),
        ],
    )(p2d, g3, target, acc_in, z_in)


def kernel(preds, target, table):
    b, d, s = preds.shape
    bc = b // _NCHUNK
    nw = 32
    nch = (bc * s) // (nw * _CH)
    p2d = jnp.reshape(preds, (b, d * s))

    gs = []
    for c in range(_NCHUNK):
        idx3 = lax.slice_in_dim(target, c * bc, (c + 1) * bc, axis=0).reshape(
            nw, nch, _CH)
        gs.append(_gather_rows_sc(table, idx3).reshape(bc, s, d))

    acc = jnp.zeros((1, 128), jnp.float32)
    zin = jnp.zeros((d, d), jnp.float32)
    for c in range(_NCHUNK):
        acc, zin = _loss_tc_chunk(
            p2d, gs[c], target, acc, zin, c, bc, d, s, last=(c == _NCHUNK - 1))
    return acc[0, 0]


# speedup vs baseline: 1.0011x; 1.0011x over previous
import functools

import jax
import jax.numpy as jnp
from jax import lax
from jax.experimental import pallas as pl
from jax.experimental.pallas import tpu as pltpu
from jax.experimental.pallas import tpu_sc as plsc

_PAD_ID = 0
_REG1 = 0.01
_REG2 = 0.1
_CH = 128
_NCHUNK = 2
_BB = 128


def _gather_rows_sc(table, idx4, c):
    _, nw, nch, ch = idx4.shape
    _, d = table.shape
    n = nw * nch * ch
    info = plsc.get_sparse_core_info()
    nc = info.num_cores
    mesh = plsc.VectorSubcoreMesh(core_axis_name="c", subcore_axis_name="s")

    @functools.partial(
        pl.kernel,
        out_type=jax.ShapeDtypeStruct((n, d), jnp.float32),
        mesh=mesh,
        scratch_types=[
            pltpu.VMEM((nch, ch), jnp.int32),
            pltpu.VMEM((ch, d), jnp.float32),
            pltpu.VMEM((ch, d), jnp.float32),
            pltpu.SemaphoreType.DMA,
            pltpu.SemaphoreType.DMA,
        ],
    )
    def k(tab_hbm, idx_hbm, out_hbm, idx_v, buf0, buf1, sem0, sem1):
        wid = lax.axis_index("s") * nc + lax.axis_index("c")
        base = wid * (nch * ch)
        pltpu.sync_copy(idx_hbm.at[c, wid], idx_v)
        pltpu.async_copy(tab_hbm.at[idx_v.at[0]], buf0, sem0)

        def body(jj, carry):
            j0 = 2 * jj
            pltpu.async_copy(tab_hbm.at[idx_v.at[j0 + 1]], buf1, sem1)
            pltpu.make_async_copy(tab_hbm.at[idx_v.at[j0]], buf0, sem0).wait()
            pltpu.sync_copy(buf0, out_hbm.at[pl.ds(base + j0 * ch, ch)])

            @pl.when(2 * jj + 2 < nch)
            def _():
                pltpu.async_copy(tab_hbm.at[idx_v.at[j0 + 2]], buf0, sem0)

            pltpu.make_async_copy(tab_hbm.at[idx_v.at[j0 + 1]], buf1, sem1).wait()
            pltpu.sync_copy(buf1, out_hbm.at[pl.ds(base + (j0 + 1) * ch, ch)])
            return carry

        lax.fori_loop(0, nch // 2, body, 0)
        if nch % 2 == 1:
            jl = nch - 1
            pltpu.make_async_copy(tab_hbm.at[idx_v.at[jl]], buf0, sem0).wait()
            pltpu.sync_copy(buf0, out_hbm.at[pl.ds(base + jl * ch, ch)])

    return k(table, idx4)


def _loss_tc_chunk(p2d, g3, target, acc_in, z_in, c, bc, d, s, last):
    bb = _BB
    grid = bc // bb
    goff = c * grid

    def body(p_ref, g_ref, t_ref, ai_ref, zi_ref, out_ref, z_ref):
        i = pl.program_id(0)
        pb = p_ref[...].astype(jnp.bfloat16).reshape(bb, d, s)
        gg = g_ref[...]
        t = t_ref[...]
        m = (t != _PAD_ID).astype(jnp.float32)
        pf = pb.astype(jnp.float32)
        z2 = jnp.sum(pf * pf, axis=1)
        z = jnp.sqrt(z2)
        v = jnp.float32(float(d))
        sq = jnp.sqrt((v + 1.0) ** 2 + z2)
        lc = sq - (v - 1.0) * jnp.log((v - 1.0) + sq)
        base = -lc + _REG1 * z

        part = jnp.sum(base * m)
        cnt = jnp.sum(m)

        gm = (gg * m[..., None]).astype(jnp.bfloat16)

        zp0 = lax.dot_general(
            pb[0], gm[0], (((1,), (0,)), ((), ())),
            preferred_element_type=jnp.float32,
        )
        zp1 = lax.dot_general(
            pb[1], gm[1], (((1,), (0,)), ((), ())),
            preferred_element_type=jnp.float32,
        )
        for j in range(2, bb, 2):
            zp0 = zp0 + lax.dot_general(
                pb[j], gm[j], (((1,), (0,)), ((), ())),
                preferred_element_type=jnp.float32,
            )
            zp1 = zp1 + lax.dot_general(
                pb[j + 1], gm[j + 1], (((1,), (0,)), ((), ())),
                preferred_element_type=jnp.float32,
            )
        zstep = zp0 + zp1

        lane = lax.broadcasted_iota(jnp.int32, (1, 128), 1)
        upd = jnp.where(lane == 0, part, 0.0) + jnp.where(lane == 1, cnt, 0.0)

        @pl.when(i == 0)
        def _init():
            out_ref[...] = ai_ref[...]
            z_ref[...] = zi_ref[...]

        acc = out_ref[...] + upd
        out_ref[...] = acc
        zacc = z_ref[...] + zstep
        z_ref[...] = zacc

        if last:
            @pl.when(i == grid - 1)
            def _fin():
                eye = (
                    lax.broadcasted_iota(jnp.int32, (d, d), 0)
                    == lax.broadcasted_iota(jnp.int32, (d, d), 1)
                ).astype(jnp.float32)
                dotsum = jnp.sum(zacc * eye)
                tot = jnp.sum(jnp.where(lane == 0, acc, 0.0)) - _REG2 * dotsum
                den = jnp.sum(jnp.where(lane == 1, acc, 0.0))
                out_ref[...] = jnp.full((1, 128), tot / den, jnp.float32)

    return pl.pallas_call(
        body,
        grid=(grid,),
        in_specs=[
            pl.BlockSpec((bb, d * s), lambda i: (goff + i, 0)),
            pl.BlockSpec((bb, s, d), lambda i: (i, 0, 0)),
            pl.BlockSpec((bb, s), lambda i: (goff + i, 0)),
            pl.BlockSpec((1, 128), lambda i: (0, 0)),
            pl.BlockSpec((d, d), lambda i: (0, 0)),
        ],
        out_specs=[
            pl.BlockSpec((1, 128), lambda i: (0, 0)),
            pl.BlockSpec((d, d), lambda i: (0, 0)),
        ],
        out_shape=[
            jax.ShapeDtypeStruct((1, 128), jnp.float32),
            jax.ShapeDtypeStruct((d, d), jnp.float32),
        ],
    )(p2d, g3, target, acc_in, z_in)


def kernel(preds, target, table):
    b, d, s = preds.shape
    bc = b // _NCHUNK
    nw = 32
    nch = (bc * s) // (nw * _CH)
    p2d = jnp.reshape(preds, (b, d * s))

    idx4 = target.reshape(_NCHUNK, nw, nch, _CH)
    gs = []
    for c in range(_NCHUNK):
        gs.append(_gather_rows_sc(table, idx4, c).reshape(bc, s, d))

    acc = jnp.zeros((1, 128), jnp.float32)
    zin = jnp.zeros((d, d), jnp.float32)
    for c in range(_NCHUNK):
        acc, zin = _loss_tc_chunk(
            p2d, gs[c], target, acc, zin, c, bc, d, s, last=(c == _NCHUNK - 1))
    return acc[0, 0]


# speedup vs baseline: 1.4312x; 1.4295x over previous
import functools

import jax
import jax.numpy as jnp
from jax import lax
from jax.experimental import pallas as pl
from jax.experimental.pallas import tpu as pltpu
from jax.experimental.pallas import tpu_sc as plsc

_PAD_ID = 0
_REG1 = 0.01
_REG2 = 0.1
_CH = 128
_NCHUNK = 2
_BB = 128


def _gather_rows_sc(table, idx4, c):
    _, nw, nch, ch = idx4.shape
    _, d = table.shape
    n = nw * nch * ch
    info = plsc.get_sparse_core_info()
    nc = info.num_cores
    mesh = plsc.VectorSubcoreMesh(core_axis_name="c", subcore_axis_name="s")

    @functools.partial(
        pl.kernel,
        out_type=jax.ShapeDtypeStruct((n, d), jnp.float32),
        mesh=mesh,
        scratch_types=[
            pltpu.VMEM((nch, ch), jnp.int32),
            pltpu.VMEM((ch, d), jnp.float32),
            pltpu.VMEM((ch, d), jnp.float32),
            pltpu.SemaphoreType.DMA,
            pltpu.SemaphoreType.DMA,
        ],
    )
    def k(tab_hbm, idx_hbm, out_hbm, idx_v, buf0, buf1, sem0, sem1):
        wid = lax.axis_index("s") * nc + lax.axis_index("c")
        base = wid * (nch * ch)
        pltpu.sync_copy(idx_hbm.at[c, wid], idx_v)
        pltpu.async_copy(tab_hbm.at[idx_v.at[0]], buf0, sem0)

        def body(jj, carry):
            j0 = 2 * jj
            pltpu.async_copy(tab_hbm.at[idx_v.at[j0 + 1]], buf1, sem1)
            pltpu.make_async_copy(tab_hbm.at[idx_v.at[j0]], buf0, sem0).wait()
            pltpu.sync_copy(buf0, out_hbm.at[pl.ds(base + j0 * ch, ch)])

            @pl.when(2 * jj + 2 < nch)
            def _():
                pltpu.async_copy(tab_hbm.at[idx_v.at[j0 + 2]], buf0, sem0)

            pltpu.make_async_copy(tab_hbm.at[idx_v.at[j0 + 1]], buf1, sem1).wait()
            pltpu.sync_copy(buf1, out_hbm.at[pl.ds(base + (j0 + 1) * ch, ch)])
            return carry

        lax.fori_loop(0, nch // 2, body, 0)
        if nch % 2 == 1:
            jl = nch - 1
            pltpu.make_async_copy(tab_hbm.at[idx_v.at[jl]], buf0, sem0).wait()
            pltpu.sync_copy(buf0, out_hbm.at[pl.ds(base + jl * ch, ch)])

    return k(table, idx4)


def _loss_tc_chunk(pt3, g3, target, acc_in, c, bc, d, s, last):
    bb = _BB
    grid = bc // bb
    goff = c * grid

    def body(p_ref, g_ref, t_ref, ai_ref, out_ref):
        i = pl.program_id(0)
        pf = p_ref[...]
        gg = g_ref[...]
        t = t_ref[...]
        m = (t != _PAD_ID).astype(jnp.float32)
        z2 = jnp.sum(pf * pf, axis=2)
        z = jnp.sqrt(z2)
        v = jnp.float32(float(d))
        sq = jnp.sqrt((v + 1.0) ** 2 + z2)
        lc = sq - (v - 1.0) * jnp.log((v - 1.0) + sq)
        base = -lc + _REG1 * z

        gm = gg * m[..., None]
        part = jnp.sum(base * m) - _REG2 * jnp.sum(pf * gm)
        cnt = jnp.sum(m)

        lane = lax.broadcasted_iota(jnp.int32, (1, 128), 1)
        upd = jnp.where(lane == 0, part, 0.0) + jnp.where(lane == 1, cnt, 0.0)

        @pl.when(i == 0)
        def _init():
            out_ref[...] = ai_ref[...]

        acc = out_ref[...] + upd
        out_ref[...] = acc

        if last:
            @pl.when(i == grid - 1)
            def _fin():
                tot = jnp.sum(jnp.where(lane == 0, acc, 0.0))
                den = jnp.sum(jnp.where(lane == 1, acc, 0.0))
                out_ref[...] = jnp.full((1, 128), tot / den, jnp.float32)

    return pl.pallas_call(
        body,
        grid=(grid,),
        in_specs=[
            pl.BlockSpec((bb, s, d), lambda i: (goff + i, 0, 0)),
            pl.BlockSpec((bb, s, d), lambda i: (i, 0, 0)),
            pl.BlockSpec((bb, s), lambda i: (goff + i, 0)),
            pl.BlockSpec((1, 128), lambda i: (0, 0)),
        ],
        out_specs=pl.BlockSpec((1, 128), lambda i: (0, 0)),
        out_shape=jax.ShapeDtypeStruct((1, 128), jnp.float32),
    )(pt3, g3, target, acc_in)


def kernel(preds, target, table):
    b, d, s = preds.shape
    bc = b // _NCHUNK
    nw = 32
    nch = (bc * s) // (nw * _CH)
    pt3 = jnp.transpose(preds, (0, 2, 1))

    idx4 = target.reshape(_NCHUNK, nw, nch, _CH)
    gs = []
    for c in range(_NCHUNK):
        gs.append(_gather_rows_sc(table, idx4, c).reshape(bc, s, d))

    acc = jnp.zeros((1, 128), jnp.float32)
    for c in range(_NCHUNK):
        acc = _loss_tc_chunk(
            pt3, gs[c], target, acc, c, bc, d, s, last=(c == _NCHUNK - 1))
    return acc[0, 0]
